# baseline (speedup 1.0000x reference)
import jax
import jax.numpy as jnp
from jax import lax
from jax.experimental import pallas as pl
from jax.experimental.pallas import tpu as pltpu
from jax.experimental.pallas import tpu_sc as plsc

T_ = 26
B_ = 4096
BAG_ = 20
V_ = 100000
D_ = 32
N_ = T_ * B_
E_ = 1048576
NC, NS, L_ = 2, 16, 16
HALF = N_ // 2
RPT = HALF // NS
PCH = 80
FBLK = PCH * 32
ECH = 128
EBLK = 2048
ZCH = 64
DUMMY = 2048

_mesh = plsc.VectorSubcoreMesh(core_axis_name="c", subcore_axis_name="s")
_sc_params = pltpu.CompilerParams(use_tc_tiling_on_sc=False)
_sc_params_nl = pltpu.CompilerParams(use_tc_tiling_on_sc=False,
                                     needs_layout_passes=False)


def _zero_acc(zb, acc, s, nrows_per_tile):
    z = jnp.zeros((L_,), jnp.float32)

    def zrow(r, carry):
        zb[r, pl.ds(0, L_)] = z
        zb[r, pl.ds(L_, L_)] = z
        return carry

    lax.fori_loop(0, ZCH, zrow, 0)
    for j in range(nrows_per_tile // ZCH):
        pltpu.sync_copy(zb, acc.at[pl.ds(s * nrows_per_tile + j * ZCH, ZCH)])


def _poolT_body(featT_hbm, tabT_hbm, xT_hbm, col, fb0, fb1, outc,
                sem, sf0, sf1):
    c = lax.axis_index("c")
    s = lax.axis_index("s")
    TPC = T_ // NC
    CHB = 512
    nch = B_ // CHB

    def one(tg, d):
        pltpu.sync_copy(tabT_hbm.at[tg, d], col)

        def stage(cb, fb, sf):
            return pltpu.async_copy(
                featT_hbm.at[tg, :, pl.ds(cb * CHB, CHB)], fb, sf)

        def chunk(cb, fb):
            for blk in range(CHB // L_):
                acc0 = jnp.zeros((L_,), jnp.float32)
                acc1 = jnp.zeros((L_,), jnp.float32)
                for j in range(0, BAG_, 2):
                    i0 = fb[j, pl.ds(blk * L_, L_)]
                    i1 = fb[j + 1, pl.ds(blk * L_, L_)]
                    acc0 = acc0 + plsc.load_gather(col, [i0])
                    acc1 = acc1 + plsc.load_gather(col, [i1])
                outc[pl.ds(cb * CHB + blk * L_, L_)] = acc0 + acc1

        stage(0, fb0, sf0)

        def pair(p_, carry):
            i0 = 2 * p_
            i1 = i0 + 1
            stage(i1, fb1, sf1)
            pltpu.make_async_copy(
                featT_hbm.at[tg, :, pl.ds(i0 * CHB, CHB)], fb0, sf0).wait()
            chunk(i0, fb0)

            @pl.when(i0 + 2 < nch)
            def _():
                stage(i0 + 2, fb0, sf0)

            pltpu.make_async_copy(
                featT_hbm.at[tg, :, pl.ds(i1 * CHB, CHB)], fb1, sf1).wait()
            chunk(i1, fb1)
            return carry

        lax.fori_loop(0, nch // 2, pair, 0)
        pltpu.sync_copy(outc, xT_hbm.at[d, pl.ds(tg * B_, B_)])

    def table_loop(ti, carry):
        tg = c * TPC + ti
        one(tg, 2 * s)
        one(tg, 2 * s + 1)
        return carry

    lax.fori_loop(0, TPC, table_loop, 0)


def _edge_body(src_hbm, dst_hbm, x_hbm, agg_hbm, sblk, dblk, gi0, gi1,
               oi0, oi1, rows0, rows1, zb, acc, sem0, sem1):
    c = lax.axis_index("c")
    s = lax.axis_index("s")
    _zero_acc(zb, acc, s, RPT)
    plsc.subcore_barrier()

    ept = E_ // NS
    e_base = s * ept
    iota = lax.iota(jnp.int32, L_)
    lo = c * HALF

    def prep(j, i, gi, oi):
        for k in range(ECH // L_):
            off = i * ECH + k * L_
            gi[pl.ds(k * L_, L_)] = sblk[pl.ds(off, L_)]
            d = dblk[pl.ds(off, L_)] - lo
            ok = (d >= 0) & (d < HALF)
            dummy = HALF + ((iota + (j * 8 + k) * L_) & (DUMMY - 1))
            oi[pl.ds(k * L_, L_)] = jnp.where(ok, d, dummy)

    def block(j, carry):
        pltpu.sync_copy(src_hbm.at[pl.ds(e_base + j * EBLK, EBLK)], sblk)
        pltpu.sync_copy(dst_hbm.at[pl.ds(e_base + j * EBLK, EBLK)], dblk)
        nchunk = EBLK // ECH

        prep(j, 0, gi0, oi0)
        pltpu.async_copy(x_hbm.at[gi0], rows0, sem0)

        def pair(p_, carry2):
            i0 = 2 * p_
            i1 = i0 + 1
            prep(j, i1, gi1, oi1)
            pltpu.async_copy(x_hbm.at[gi1], rows1, sem1)
            pltpu.make_async_copy(x_hbm.at[gi0], rows0, sem0).wait()
            pltpu.sync_copy(rows0, acc.at[oi0], add=True)

            @pl.when(i0 + 2 < nchunk)
            def _():
                prep(j, i0 + 2, gi0, oi0)
                pltpu.async_copy(x_hbm.at[gi0], rows0, sem0)

            pltpu.make_async_copy(x_hbm.at[gi1], rows1, sem1).wait()
            pltpu.sync_copy(rows1, acc.at[oi1], add=True)
            return carry2

        lax.fori_loop(0, nchunk // 2, pair, 0)
        return carry

    lax.fori_loop(0, ept // EBLK, block, 0)
    plsc.subcore_barrier()
    pltpu.sync_copy(acc.at[pl.ds(s * RPT, RPT)],
                    agg_hbm.at[pl.ds(c * HALF + s * RPT, RPT)])


@jax.jit
def _poolT(featT, tabT):
    return pl.kernel(
        _poolT_body,
        out_type=jax.ShapeDtypeStruct((D_, N_), jnp.float32),
        mesh=_mesh,
        compiler_params=_sc_params_nl,
        scratch_types=[
            pltpu.VMEM((V_,), jnp.float32),
            pltpu.VMEM((BAG_, 512), jnp.int32),
            pltpu.VMEM((BAG_, 512), jnp.int32),
            pltpu.VMEM((B_,), jnp.float32),
            pltpu.SemaphoreType.DMA,
            pltpu.SemaphoreType.DMA,
            pltpu.SemaphoreType.DMA,
        ],
    )(featT, tabT)


@jax.jit
def _agg(src, dst, x):
    return pl.kernel(
        _edge_body,
        out_type=jax.ShapeDtypeStruct((N_, D_), jnp.float32),
        mesh=_mesh,
        compiler_params=_sc_params,
        scratch_types=[
            pltpu.VMEM((EBLK,), jnp.int32),
            pltpu.VMEM((EBLK,), jnp.int32),
            pltpu.VMEM((ECH,), jnp.int32),
            pltpu.VMEM((ECH,), jnp.int32),
            pltpu.VMEM((ECH,), jnp.int32),
            pltpu.VMEM((ECH,), jnp.int32),
            pltpu.VMEM((ECH, D_), jnp.float32),
            pltpu.VMEM((ECH, D_), jnp.float32),
            pltpu.VMEM((ZCH, D_), jnp.float32),
            pltpu.VMEM_SHARED((HALF + DUMMY, D_), jnp.float32),
            pltpu.SemaphoreType.DMA,
            pltpu.SemaphoreType.DMA,
        ],
    )(src, dst, x)


def _update_body(a_ref, x_ref, w_ref, b_ref, o_ref):
    acc = jnp.dot(a_ref[...], w_ref[...], preferred_element_type=jnp.float32)
    o_ref[...] = jnp.maximum(acc + b_ref[...] + x_ref[...], 0.0)


ROWBLK = 2048


@jax.jit
def _update(agg, x, W, b2):
    grid = (N_ // ROWBLK,)
    return pl.pallas_call(
        _update_body,
        grid=grid,
        in_specs=[
            pl.BlockSpec((ROWBLK, D_), lambda i: (i, 0)),
            pl.BlockSpec((ROWBLK, D_), lambda i: (i, 0)),
            pl.BlockSpec((D_, D_), lambda i: (0, 0)),
            pl.BlockSpec((1, D_), lambda i: (0, 0)),
        ],
        out_specs=pl.BlockSpec((ROWBLK, D_), lambda i: (i, 0)),
        out_shape=jax.ShapeDtypeStruct((N_, D_), jnp.float32),
    )(agg, x, W, b2)


def kernel(features, edge_index, tables, W, b):
    featT = features.transpose(0, 2, 1).astype(jnp.int32)
    src = edge_index[0].astype(jnp.int32)
    dst = edge_index[1].astype(jnp.int32)
    tabT = tables.transpose(0, 2, 1)
    xT = _poolT(featT, tabT)
    x = xT.T
    agg = _agg(src, dst, x)
    h = _update(agg, x, W, b.reshape(1, D_))
    return h.reshape(T_, B_, D_)


# speedup vs baseline: 1.0868x; 1.0868x over previous
import jax
import jax.numpy as jnp
from jax import lax
from jax.experimental import pallas as pl
from jax.experimental.pallas import tpu as pltpu
from jax.experimental.pallas import tpu_sc as plsc

T_ = 26
B_ = 4096
BAG_ = 20
V_ = 100000
D_ = 32
N_ = T_ * B_
E_ = 1048576
NC, NS, L_ = 2, 16, 16
HALF = N_ // 2
RPT = HALF // NS
PCH = 80
FBLK = PCH * 32
ECH = 128
EBLK = 2048
ZCH = 64
DUMMY = 2048

_mesh = plsc.VectorSubcoreMesh(core_axis_name="c", subcore_axis_name="s")
_sc_params = pltpu.CompilerParams(use_tc_tiling_on_sc=False)
_sc_params_nl = pltpu.CompilerParams(use_tc_tiling_on_sc=False,
                                     needs_layout_passes=False)


def _zero_acc(zb, acc, s, nrows_per_tile):
    z = jnp.zeros((L_,), jnp.float32)

    def zrow(r, carry):
        zb[r, pl.ds(0, L_)] = z
        zb[r, pl.ds(L_, L_)] = z
        return carry

    lax.fori_loop(0, ZCH, zrow, 0)
    for j in range(nrows_per_tile // ZCH):
        pltpu.sync_copy(zb, acc.at[pl.ds(s * nrows_per_tile + j * ZCH, ZCH)])


def _poolT_body(ntab, featT_hbm, tabT_hbm, xT_hbm, col, fb0, fb1, outc,
                sem, sf0, sf1):
    c = lax.axis_index("c")
    s = lax.axis_index("s")
    TPC = ntab // NC
    CHB = 256
    nch = B_ // CHB

    def one(tg, d):
        pltpu.sync_copy(tabT_hbm.at[tg, d], col)

        def stage(cb, fb, sf):
            return pltpu.async_copy(
                featT_hbm.at[tg, :, pl.ds(cb * CHB, CHB)], fb, sf)

        def chunk(cb, fb):
            for blk in range(CHB // L_):
                acc = jnp.zeros((L_,), jnp.float32)
                for j in range(BAG_):
                    idx = fb[j, pl.ds(blk * L_, L_)]
                    acc = acc + plsc.load_gather(col, [idx])
                outc[pl.ds(cb * CHB + blk * L_, L_)] = acc

        stage(0, fb0, sf0)

        def pair(p_, carry):
            i0 = 2 * p_
            i1 = i0 + 1
            stage(i1, fb1, sf1)
            pltpu.make_async_copy(
                featT_hbm.at[tg, :, pl.ds(i0 * CHB, CHB)], fb0, sf0).wait()
            chunk(i0, fb0)

            @pl.when(i0 + 2 < nch)
            def _():
                stage(i0 + 2, fb0, sf0)

            pltpu.make_async_copy(
                featT_hbm.at[tg, :, pl.ds(i1 * CHB, CHB)], fb1, sf1).wait()
            chunk(i1, fb1)
            return carry

        lax.fori_loop(0, nch // 2, pair, 0)
        pltpu.sync_copy(outc, xT_hbm.at[d, pl.ds(tg * B_, B_)])

    def table_loop(ti, carry):
        tg = c * TPC + ti
        one(tg, 2 * s)
        one(tg, 2 * s + 1)
        return carry

    lax.fori_loop(0, TPC, table_loop, 0)


def _edge_body(src_hbm, dst_hbm, x_hbm, agg_hbm, sblk, dblk, gi0, gi1,
               oi0, oi1, rows0, rows1, zb, acc, sem0, sem1):
    c = lax.axis_index("c")
    s = lax.axis_index("s")
    _zero_acc(zb, acc, s, RPT)
    plsc.subcore_barrier()

    ept = E_ // NS
    e_base = s * ept
    iota = lax.iota(jnp.int32, L_)
    lo = c * HALF

    def prep(j, i, gi, oi):
        for k in range(ECH // L_):
            off = i * ECH + k * L_
            gi[pl.ds(k * L_, L_)] = sblk[pl.ds(off, L_)]
            d = dblk[pl.ds(off, L_)] - lo
            ok = (d >= 0) & (d < HALF)
            dummy = HALF + ((iota + (j * 8 + k) * L_) & (DUMMY - 1))
            oi[pl.ds(k * L_, L_)] = jnp.where(ok, d, dummy)

    def block(j, carry):
        pltpu.sync_copy(src_hbm.at[pl.ds(e_base + j * EBLK, EBLK)], sblk)
        pltpu.sync_copy(dst_hbm.at[pl.ds(e_base + j * EBLK, EBLK)], dblk)
        nchunk = EBLK // ECH

        prep(j, 0, gi0, oi0)
        pltpu.async_copy(x_hbm.at[gi0], rows0, sem0)

        def pair(p_, carry2):
            i0 = 2 * p_
            i1 = i0 + 1
            prep(j, i1, gi1, oi1)
            pltpu.async_copy(x_hbm.at[gi1], rows1, sem1)
            pltpu.make_async_copy(x_hbm.at[gi0], rows0, sem0).wait()
            pltpu.sync_copy(rows0, acc.at[oi0], add=True)

            @pl.when(i0 + 2 < nchunk)
            def _():
                prep(j, i0 + 2, gi0, oi0)
                pltpu.async_copy(x_hbm.at[gi0], rows0, sem0)

            pltpu.make_async_copy(x_hbm.at[gi1], rows1, sem1).wait()
            pltpu.sync_copy(rows1, acc.at[oi1], add=True)
            return carry2

        lax.fori_loop(0, nchunk // 2, pair, 0)
        return carry

    lax.fori_loop(0, ept // EBLK, block, 0)
    plsc.subcore_barrier()
    pltpu.sync_copy(acc.at[pl.ds(s * RPT, RPT)],
                    agg_hbm.at[pl.ds(c * HALF + s * RPT, RPT)])


import functools


@functools.partial(jax.jit, static_argnums=0)
def _poolT(ntab, featT, tabT):
    return pl.kernel(
        functools.partial(_poolT_body, ntab),
        out_type=jax.ShapeDtypeStruct((D_, ntab * B_), jnp.float32),
        mesh=_mesh,
        compiler_params=_sc_params_nl,
        scratch_types=[
            pltpu.VMEM((V_,), jnp.float32),
            pltpu.VMEM((BAG_, 256), jnp.int32),
            pltpu.VMEM((BAG_, 256), jnp.int32),
            pltpu.VMEM((B_,), jnp.float32),
            pltpu.SemaphoreType.DMA,
            pltpu.SemaphoreType.DMA,
            pltpu.SemaphoreType.DMA,
        ],
    )(featT, tabT)


@jax.jit
def _agg(src, dst, x):
    return pl.kernel(
        _edge_body,
        out_type=jax.ShapeDtypeStruct((N_, D_), jnp.float32),
        mesh=_mesh,
        compiler_params=_sc_params,
        scratch_types=[
            pltpu.VMEM((EBLK,), jnp.int32),
            pltpu.VMEM((EBLK,), jnp.int32),
            pltpu.VMEM((ECH,), jnp.int32),
            pltpu.VMEM((ECH,), jnp.int32),
            pltpu.VMEM((ECH,), jnp.int32),
            pltpu.VMEM((ECH,), jnp.int32),
            pltpu.VMEM((ECH, D_), jnp.float32),
            pltpu.VMEM((ECH, D_), jnp.float32),
            pltpu.VMEM((ZCH, D_), jnp.float32),
            pltpu.VMEM_SHARED((HALF + DUMMY, D_), jnp.float32),
            pltpu.SemaphoreType.DMA,
            pltpu.SemaphoreType.DMA,
        ],
    )(src, dst, x)


def _update_body(a_ref, x_ref, w_ref, b_ref, o_ref):
    acc = jnp.dot(a_ref[...], w_ref[...], preferred_element_type=jnp.float32)
    o_ref[...] = jnp.maximum(acc + b_ref[...] + x_ref[...], 0.0)


ROWBLK = 2048


@jax.jit
def _update(agg, x, W, b2):
    grid = (N_ // ROWBLK,)
    return pl.pallas_call(
        _update_body,
        grid=grid,
        in_specs=[
            pl.BlockSpec((ROWBLK, D_), lambda i: (i, 0)),
            pl.BlockSpec((ROWBLK, D_), lambda i: (i, 0)),
            pl.BlockSpec((D_, D_), lambda i: (0, 0)),
            pl.BlockSpec((1, D_), lambda i: (0, 0)),
        ],
        out_specs=pl.BlockSpec((ROWBLK, D_), lambda i: (i, 0)),
        out_shape=jax.ShapeDtypeStruct((N_, D_), jnp.float32),
    )(agg, x, W, b2)


def kernel(features, edge_index, tables, W, b):
    featT = features.transpose(0, 2, 1).astype(jnp.int32)
    src = edge_index[0].astype(jnp.int32)
    dst = edge_index[1].astype(jnp.int32)
    tabT = tables.transpose(0, 2, 1)
    xTs = []
    lo = 0
    for ntab in (8, 6, 6, 6):
        f_g = lax.slice_in_dim(featT, lo, lo + ntab, axis=0)
        t_g = lax.slice_in_dim(tabT, lo, lo + ntab, axis=0)
        xTs.append(_poolT(ntab, f_g, t_g))
        lo += ntab
    xT = jnp.concatenate(xTs, axis=1)
    x = xT.T
    agg = _agg(src, dst, x)
    h = _update(agg, x, W, b.reshape(1, D_))
    return h.reshape(T_, B_, D_)
